# baseline (speedup 1.0000x reference)
import jax
import jax.numpy as jnp
from jax import lax
from jax.experimental import pallas as pl
from jax.experimental.pallas import tpu as pltpu
from jax.experimental.pallas import tpu_sc as plsc

_S = 512
_L = 16
_W = 144
_G = 128


def _sc_body(aug_hbm, xt_hbm, yt_hbm, zt_hbm, bprep_hbm,
             pooled_hbm, flag_hbm,
             xv, yv, zv, bx, idx_build, idx1d, fbuf0, fbuf1, obuf0, obuf1,
             flags_v, gsem, osem):
  B, N = xt_hbm.shape
  boxes_per_w = 16
  n_chunks = N // _L
  zero_row = B * N

  wid = lax.axis_index("s") * 2 + lax.axis_index("c")
  b = wid // 8
  m_base = (wid % 8) * boxes_per_w

  pltpu.sync_copy(xt_hbm.at[b], xv)
  pltpu.sync_copy(yt_hbm.at[b], yv)
  pltpu.sync_copy(zt_hbm.at[b], zv)
  pltpu.sync_copy(bprep_hbm.at[pl.ds(wid * boxes_per_w, boxes_per_w)], bx)

  lane = lax.iota(jnp.int32, _L)

  def one_box(j, carry):
    m = m_base + j
    brow = bx[j, :]
    cx = brow[0]
    cy = brow[1]
    czc = brow[2]
    dx2 = brow[3]
    dy2 = brow[4]
    dz2 = brow[5]
    cosa = brow[6]
    sina = brow[7]

    def in_box(i):
      x = xv[pl.ds(i * _L, _L)]
      y = yv[pl.ds(i * _L, _L)]
      z = zv[pl.ds(i * _L, _L)]
      z_ok = jnp.abs(z - czc) <= dz2
      sx = x - cx
      sy = y - cy
      local_x = sx * cosa + sy * (-sina)
      local_y = sx * sina + sy * cosa
      return (z_ok & (local_x > -dx2) & (local_x < dx2)
              & (local_y > -dy2) & (local_y < dy2))

    def cnt_body(i, counts):
      return counts + in_box(i).astype(jnp.int32)

    counts = lax.fori_loop(0, n_chunks, cnt_body,
                           jnp.zeros((_L,), jnp.int32), unroll=16)
    incl = jnp.cumsum(counts)
    cnt = incl[_L - 1]
    ex = incl - counts
    lanebase = lane * n_chunks + b * N

    def wr_body(i, ptrs):
      mask = in_box(i)
      wr = mask & (ptrs < _S)
      plsc.store_scatter(idx_build, (ptrs,), lanebase + i, mask=wr)
      return ptrs + mask.astype(jnp.int32)

    lax.fori_loop(0, n_chunks, wr_body, ex, unroll=16)

    empty = cnt == 0
    flags_v[...] = jnp.where(lane == j, empty.astype(jnp.int32), flags_v[...])

    denom = jnp.maximum(cnt, 1)
    for cch in range(_S // _L):
      pos = (lane + cch * _L) % denom
      vals = plsc.load_gather(idx_build, (pos,))
      vals = jnp.where(empty, zero_row, vals)
      idx1d[pl.ds(cch * _L, _L)] = vals

    tail_mask = lane < 3
    fbufs = (fbuf0, fbuf1)
    obufs = (obuf0, obuf1)
    nch = _S // _G
    gathers = [None] * nch
    outcps = [None] * nch

    def start_gather(g):
      gathers[g] = pltpu.async_copy(
          aug_hbm.at[idx1d.at[pl.ds(g * _G, _G)]], fbufs[g % 2], gsem)

    start_gather(0)
    for g in range(nch):
      fbuf = fbufs[g % 2]
      obuf = obufs[g % 2]
      if g >= 2:
        outcps[g - 2].wait()
      gathers[g].wait()
      if g + 1 < nch:
        start_gather(g + 1)

      def repack(r, carry2):
        rr = lane * 0 + r
        for k in range(8):
          v = fbuf[r, pl.ds(k * _L, _L)]
          plsc.store_scatter(obuf, (rr, lane + k * _L), v)
        v = fbuf[r, pl.ds(8 * _L, _L)]
        cols = jnp.minimum(lane + 8 * _L, 130)
        plsc.store_scatter(obuf, (rr, cols), v, mask=tail_mask)
        return carry2

      lax.fori_loop(0, _G, repack, 0, unroll=8)
      outcps[g] = pltpu.async_copy(
          obuf, pooled_hbm.at[b, m, pl.ds(g * _G, _G)], osem)
    outcps[nch - 2].wait()
    outcps[nch - 1].wait()
    return carry

  lax.fori_loop(0, boxes_per_w, one_box, 0)
  pltpu.sync_copy(flags_v, flag_hbm.at[b, pl.ds(m_base, boxes_per_w)])


@jax.jit
def kernel(points, point_features, boxes3d):
  B, N, _ = points.shape
  M = boxes3d.shape[1]
  C = point_features.shape[2]

  pad = jnp.zeros((B, N, _W - 3 - C), jnp.float32)
  aug = jnp.concatenate([points, point_features, pad], axis=-1)
  aug = aug.reshape(B * N, _W)
  aug = jnp.concatenate([aug, jnp.zeros((1, _W), jnp.float32)], axis=0)

  def stripes(a):
    return a.reshape(B, _L, N // _L).transpose(0, 2, 1).reshape(B, N)

  xt = stripes(points[:, :, 0])
  yt = stripes(points[:, :, 1])
  zt = stripes(points[:, :, 2])

  cx = boxes3d[:, :, 0]
  cy = boxes3d[:, :, 1]
  dz = boxes3d[:, :, 5]
  czc = boxes3d[:, :, 2] + dz / 2.0
  dx2 = boxes3d[:, :, 3] / 2.0
  dy2 = boxes3d[:, :, 4] / 2.0
  dz2 = dz / 2.0
  rz = boxes3d[:, :, 6]
  cosa = jnp.cos(-rz)
  sina = jnp.sin(-rz)
  zeros = jnp.zeros_like(cx)
  bprep = jnp.stack([cx, cy, czc, dx2, dy2, dz2, cosa, sina] + [zeros] * 8,
                    axis=-1).reshape(B * M, 16)

  mesh = plsc.VectorSubcoreMesh(core_axis_name="c", subcore_axis_name="s")
  run = pl.kernel(
      _sc_body,
      compiler_params=pltpu.CompilerParams(
          needs_layout_passes=False, use_tc_tiling_on_sc=False),
      out_type=[
          jax.ShapeDtypeStruct((B, M, _S, 3 + C), jnp.float32),
          jax.ShapeDtypeStruct((B, M), jnp.int32),
      ],
      mesh=mesh,
      scratch_types=[
          pltpu.VMEM((N,), jnp.float32),
          pltpu.VMEM((N,), jnp.float32),
          pltpu.VMEM((N,), jnp.float32),
          pltpu.VMEM((16, 16), jnp.float32),
          pltpu.VMEM((_S + 2 * _L,), jnp.int32),
          pltpu.VMEM((_S,), jnp.int32),
          pltpu.VMEM((_G, _W), jnp.float32),
          pltpu.VMEM((_G, _W), jnp.float32),
          pltpu.VMEM((_G, 131), jnp.float32),
          pltpu.VMEM((_G, 131), jnp.float32),
          pltpu.VMEM((_L,), jnp.int32),
          pltpu.SemaphoreType.DMA,
          pltpu.SemaphoreType.DMA,
      ],
  )
  pooled, flag = run(aug, xt, yt, zt, bprep)
  return pooled, flag


# speedup vs baseline: 1.0257x; 1.0257x over previous
import jax
import jax.numpy as jnp
from jax import lax
from jax.experimental import pallas as pl
from jax.experimental.pallas import tpu as pltpu
from jax.experimental.pallas import tpu_sc as plsc

_S = 512
_L = 16
_W = 144
_G = 128


def _sc_body(aug_hbm, xt_hbm, yt_hbm, zt_hbm, bprep_hbm,
             pooled_hbm, flag_hbm,
             xv, yv, zv, bx, idx_build, idx1d, fbuf0, fbuf1, obuf0, obuf1,
             flags_v, gsem, osem):
  B, N = xt_hbm.shape
  boxes_per_w = 16
  n_chunks = N // _L
  zero_row = B * N

  wid = lax.axis_index("s") * 2 + lax.axis_index("c")
  b = wid // 8
  m_base = (wid % 8) * boxes_per_w

  pltpu.sync_copy(xt_hbm.at[b], xv)
  pltpu.sync_copy(yt_hbm.at[b], yv)
  pltpu.sync_copy(zt_hbm.at[b], zv)
  pltpu.sync_copy(bprep_hbm.at[pl.ds(wid * boxes_per_w, boxes_per_w)], bx)

  lane = lax.iota(jnp.int32, _L)

  def one_box(j, carry):
    m = m_base + j
    brow = bx[j, :]
    cx = brow[0]
    cy = brow[1]
    czc = brow[2]
    dx2 = brow[3]
    dy2 = brow[4]
    dz2 = brow[5]
    cosa = brow[6]
    sina = brow[7]

    def in_box(i):
      x = xv[pl.ds(i * _L, _L)]
      y = yv[pl.ds(i * _L, _L)]
      z = zv[pl.ds(i * _L, _L)]
      z_ok = jnp.abs(z - czc) <= dz2
      sx = x - cx
      sy = y - cy
      local_x = sx * cosa + sy * (-sina)
      local_y = sx * sina + sy * cosa
      return (z_ok & (local_x > -dx2) & (local_x < dx2)
              & (local_y > -dy2) & (local_y < dy2))

    def cnt_body(i, counts):
      return counts + in_box(i).astype(jnp.int32)

    counts = lax.fori_loop(0, n_chunks, cnt_body,
                           jnp.zeros((_L,), jnp.int32), unroll=4)
    incl = jnp.cumsum(counts)
    cnt = incl[_L - 1]
    ex = incl - counts
    lanebase = lane * n_chunks + b * N

    def wr_body(i, ptrs):
      mask = in_box(i)
      wr = mask & (ptrs < _S)
      plsc.store_scatter(idx_build, (ptrs,), lanebase + i, mask=wr)
      return ptrs + mask.astype(jnp.int32)

    lax.fori_loop(0, n_chunks, wr_body, ex, unroll=4)

    empty = cnt == 0
    flags_v[...] = jnp.where(lane == j, empty.astype(jnp.int32), flags_v[...])

    denom = jnp.maximum(cnt, 1)
    for cch in range(_S // _L):
      pos = (lane + cch * _L) % denom
      vals = plsc.load_gather(idx_build, (pos,))
      vals = jnp.where(empty, zero_row, vals)
      idx1d[pl.ds(cch * _L, _L)] = vals

    tail_mask = lane < 3
    fbufs = (fbuf0, fbuf1)
    obufs = (obuf0, obuf1)
    nch = _S // _G
    gathers = [None] * nch
    outcps = [None] * nch

    def start_gather(g):
      gathers[g] = pltpu.async_copy(
          aug_hbm.at[idx1d.at[pl.ds(g * _G, _G)]], fbufs[g % 2], gsem)

    start_gather(0)
    for g in range(nch):
      fbuf = fbufs[g % 2]
      obuf = obufs[g % 2]
      if g >= 2:
        outcps[g - 2].wait()
      gathers[g].wait()
      if g + 1 < nch:
        start_gather(g + 1)

      def repack(r, carry2):
        rr = lane * 0 + r
        for k in range(8):
          v = fbuf[r, pl.ds(k * _L, _L)]
          plsc.store_scatter(obuf, (rr, lane + k * _L), v)
        v = fbuf[r, pl.ds(8 * _L, _L)]
        cols = jnp.minimum(lane + 8 * _L, 130)
        plsc.store_scatter(obuf, (rr, cols), v, mask=tail_mask)
        return carry2

      lax.fori_loop(0, _G, repack, 0)
      outcps[g] = pltpu.async_copy(
          obuf, pooled_hbm.at[b, m, pl.ds(g * _G, _G)], osem)
    outcps[nch - 2].wait()
    outcps[nch - 1].wait()
    return carry

  lax.fori_loop(0, boxes_per_w, one_box, 0)
  pltpu.sync_copy(flags_v, flag_hbm.at[b, pl.ds(m_base, boxes_per_w)])


@jax.jit
def kernel(points, point_features, boxes3d):
  B, N, _ = points.shape
  M = boxes3d.shape[1]
  C = point_features.shape[2]

  pad = jnp.zeros((B, N, _W - 3 - C), jnp.float32)
  aug = jnp.concatenate([points, point_features, pad], axis=-1)
  aug = aug.reshape(B * N, _W)
  aug = jnp.concatenate([aug, jnp.zeros((1, _W), jnp.float32)], axis=0)

  def stripes(a):
    return a.reshape(B, _L, N // _L).transpose(0, 2, 1).reshape(B, N)

  xt = stripes(points[:, :, 0])
  yt = stripes(points[:, :, 1])
  zt = stripes(points[:, :, 2])

  cx = boxes3d[:, :, 0]
  cy = boxes3d[:, :, 1]
  dz = boxes3d[:, :, 5]
  czc = boxes3d[:, :, 2] + dz / 2.0
  dx2 = boxes3d[:, :, 3] / 2.0
  dy2 = boxes3d[:, :, 4] / 2.0
  dz2 = dz / 2.0
  rz = boxes3d[:, :, 6]
  cosa = jnp.cos(-rz)
  sina = jnp.sin(-rz)
  zeros = jnp.zeros_like(cx)
  bprep = jnp.stack([cx, cy, czc, dx2, dy2, dz2, cosa, sina] + [zeros] * 8,
                    axis=-1).reshape(B * M, 16)

  mesh = plsc.VectorSubcoreMesh(core_axis_name="c", subcore_axis_name="s")
  run = pl.kernel(
      _sc_body,
      compiler_params=pltpu.CompilerParams(
          needs_layout_passes=False, use_tc_tiling_on_sc=False),
      out_type=[
          jax.ShapeDtypeStruct((B, M, _S, 3 + C), jnp.float32),
          jax.ShapeDtypeStruct((B, M), jnp.int32),
      ],
      mesh=mesh,
      scratch_types=[
          pltpu.VMEM((N,), jnp.float32),
          pltpu.VMEM((N,), jnp.float32),
          pltpu.VMEM((N,), jnp.float32),
          pltpu.VMEM((16, 16), jnp.float32),
          pltpu.VMEM((_S + 2 * _L,), jnp.int32),
          pltpu.VMEM((_S,), jnp.int32),
          pltpu.VMEM((_G, _W), jnp.float32),
          pltpu.VMEM((_G, _W), jnp.float32),
          pltpu.VMEM((_G, 131), jnp.float32),
          pltpu.VMEM((_G, 131), jnp.float32),
          pltpu.VMEM((_L,), jnp.int32),
          pltpu.SemaphoreType.DMA,
          pltpu.SemaphoreType.DMA,
      ],
  )
  pooled, flag = run(aug, xt, yt, zt, bprep)
  return pooled, flag
